# Initial kernel scaffold; baseline (speedup 1.0000x reference)
import jax
import jax.numpy as jnp
from jax.experimental import pallas as pl


def kernel(x):
    raise NotImplementedError("write your pallas kernel here")


# speedup vs baseline: 5.9232x; 5.9232x over previous
import functools

import numpy as np

import jax
import jax.numpy as jnp
from jax.experimental import pallas as pl
from jax.experimental.pallas import tpu as pltpu

_WIN = 10
_BANDS = 6
_EPS = 1e-6
_HO = 124


def _dft_mats():
    s = np.arange(_WIN)
    k = (s + _WIN // 2) % _WIN
    w = np.arange(_WIN)
    ang = 2.0 * np.pi * np.outer(k, w) / _WIN
    th = (ang[:, None, :, None] + ang[None, :, None, :])
    th = th.reshape(_WIN * _WIN, _WIN * _WIN)
    cs = np.zeros((208, _WIN * _WIN), np.float32)
    cs[0:100] = (np.cos(th) / 10.0).astype(np.float32)
    cs[104:204] = (np.sin(th) / 10.0).astype(np.float32)
    return jnp.asarray(cs)


def _band_mat():
    lin = jnp.linspace(-1.0, 1.0, _WIN)
    yy, xx = jnp.meshgrid(lin, lin, indexing='ij')
    rr = jnp.sqrt(xx * xx + yy * yy)
    rr = rr / jnp.maximum(rr.max(), 1e-6)
    edges = jnp.linspace(0.0, 1.0, _BANDS + 1)
    masks = ((rr[None] >= edges[:-1, None, None]) &
             (rr[None] < edges[1:, None, None])).astype(jnp.float32)
    counts = jnp.maximum(masks.sum(axis=(-2, -1)), 1.0)
    bm = (masks / counts[:, None, None]).reshape(_BANDS, _WIN * _WIN)
    return jnp.concatenate([bm, jnp.zeros((2, _WIN * _WIN), jnp.float32)], axis=0)


def _body(xp_ref, cs_ref, bm_ref, out_ref, pt_ref):
    g = [[None, None], [None, None]]
    for py in range(2):
        for px in range(2):
            g[py][px] = (0.2989 * xp_ref[0, py, px, 0]
                         + 0.587 * xp_ref[0, py, px, 1]
                         + 0.114 * xp_ref[0, py, px, 2])

    for wy in range(_WIN):
        py, dy = wy % 2, wy // 2
        for wx in range(_WIN):
            px, dx = wx % 2, wx // 2
            pt_ref[wy * _WIN + wx, :, 0:_HO] = (
                g[py][px][dy:dy + _HO, dx:dx + _HO])
    pt_ref[:, :, _HO:128] = jnp.zeros((100, _HO, 128 - _HO), jnp.float32)

    cs = cs_ref[...]
    bm = bm_ref[...]
    hi = jax.lax.Precision.HIGHEST
    for hc in range(0, _HO, 8):
        ch = min(8, _HO - hc)
        ptc = pt_ref[:, hc:hc + ch, :]
        reim = jnp.einsum('fo,ohw->fhw', cs, ptc,
                          preferred_element_type=jnp.float32, precision=hi)
        re = reim[0:100]
        im = reim[104:204]
        amp = jnp.sqrt(re * re + im * im)
        stat = jnp.einsum('kf,fhw->khw', bm, amp,
                          preferred_element_type=jnp.float32, precision=hi)
        out_ref[0, :, hc:hc + ch, :] = jnp.log10(stat + _EPS)


@jax.jit
def kernel(x):
    b = x.shape[0]
    xp = x.reshape(b, 3, 128, 2, 128, 2).transpose(0, 3, 5, 1, 2, 4)
    cs = _dft_mats()
    bm = _band_mat()
    out = pl.pallas_call(
        _body,
        grid=(b,),
        in_specs=[
            pl.BlockSpec((1, 2, 2, 3, 128, 128), lambda i: (i, 0, 0, 0, 0, 0)),
            pl.BlockSpec((208, 100), lambda i: (0, 0)),
            pl.BlockSpec((8, 100), lambda i: (0, 0)),
        ],
        out_specs=pl.BlockSpec((1, 8, _HO, 128), lambda i: (i, 0, 0, 0)),
        out_shape=jax.ShapeDtypeStruct((b, 8, _HO, 128), jnp.float32),
        scratch_shapes=[pltpu.VMEM((100, _HO, 128), jnp.float32)],
        compiler_params=pltpu.CompilerParams(
            dimension_semantics=("arbitrary",),
            vmem_limit_bytes=56 * 1024 * 1024,
        ),
    )(xp, cs, bm)
    return out[:, :_BANDS, :, :_HO]


# speedup vs baseline: 10.7649x; 1.8174x over previous
import functools

import numpy as np

import jax
import jax.numpy as jnp
from jax.experimental import pallas as pl
from jax.experimental.pallas import tpu as pltpu

_WIN = 10
_BANDS = 6
_EPS = 1e-6
_HO = 124


def _dft_mats():
    s = np.arange(_WIN)
    k = (s + _WIN // 2) % _WIN
    w = np.arange(_WIN)
    ang = 2.0 * np.pi * np.outer(k, w) / _WIN
    th = (ang[:, None, :, None] + ang[None, :, None, :])
    th = th.reshape(_WIN * _WIN, _WIN * _WIN)
    cs = np.zeros((208, _WIN * _WIN), np.float32)
    cs[0:100] = (np.cos(th) / 10.0).astype(np.float32)
    cs[104:204] = (np.sin(th) / 10.0).astype(np.float32)
    return jnp.asarray(cs)


def _band_mat():
    lin = jnp.linspace(-1.0, 1.0, _WIN)
    yy, xx = jnp.meshgrid(lin, lin, indexing='ij')
    rr = jnp.sqrt(xx * xx + yy * yy)
    rr = rr / jnp.maximum(rr.max(), 1e-6)
    edges = jnp.linspace(0.0, 1.0, _BANDS + 1)
    masks = ((rr[None] >= edges[:-1, None, None]) &
             (rr[None] < edges[1:, None, None])).astype(jnp.float32)
    counts = jnp.maximum(masks.sum(axis=(-2, -1)), 1.0)
    bm = (masks / counts[:, None, None]).reshape(_BANDS, _WIN * _WIN)
    return jnp.concatenate([bm, jnp.zeros((2, _WIN * _WIN), jnp.float32)], axis=0)


def _body(xp_ref, cs_ref, bm_ref, out_ref, pt_ref):
    g = [[None, None], [None, None]]
    for py in range(2):
        for px in range(2):
            g[py][px] = (0.2989 * xp_ref[0, py, px, 0]
                         + 0.587 * xp_ref[0, py, px, 1]
                         + 0.114 * xp_ref[0, py, px, 2])

    for wy in range(_WIN):
        py, dy = wy % 2, wy // 2
        for wx in range(_WIN):
            px, dx = wx % 2, wx // 2
            pt_ref[wy * _WIN + wx, :, 0:_HO] = (
                g[py][px][dy:dy + _HO, dx:dx + _HO])
    pt_ref[:, :, _HO:128] = jnp.zeros((100, _HO, 128 - _HO), jnp.float32)

    cs = cs_ref[...]
    bm = bm_ref[...]
    hi = jax.lax.Precision.DEFAULT
    for hc in range(0, _HO, 8):
        ch = min(8, _HO - hc)
        ptc = pt_ref[:, hc:hc + ch, :]
        reim = jnp.einsum('fo,ohw->fhw', cs, ptc,
                          preferred_element_type=jnp.float32, precision=hi)
        re = reim[0:100]
        im = reim[104:204]
        amp = jnp.sqrt(re * re + im * im)
        stat = jnp.einsum('kf,fhw->khw', bm, amp,
                          preferred_element_type=jnp.float32, precision=hi)
        out_ref[0, :, hc:hc + ch, :] = jnp.log10(stat + _EPS)


@jax.jit
def kernel(x):
    b = x.shape[0]
    xp = x.reshape(b, 3, 128, 2, 128, 2).transpose(0, 3, 5, 1, 2, 4)
    cs = _dft_mats()
    bm = _band_mat()
    out = pl.pallas_call(
        _body,
        grid=(b,),
        in_specs=[
            pl.BlockSpec((1, 2, 2, 3, 128, 128), lambda i: (i, 0, 0, 0, 0, 0)),
            pl.BlockSpec((208, 100), lambda i: (0, 0)),
            pl.BlockSpec((8, 100), lambda i: (0, 0)),
        ],
        out_specs=pl.BlockSpec((1, 8, _HO, 128), lambda i: (i, 0, 0, 0)),
        out_shape=jax.ShapeDtypeStruct((b, 8, _HO, 128), jnp.float32),
        scratch_shapes=[pltpu.VMEM((100, _HO, 128), jnp.float32)],
        compiler_params=pltpu.CompilerParams(
            dimension_semantics=("arbitrary",),
            vmem_limit_bytes=56 * 1024 * 1024,
        ),
    )(xp, cs, bm)
    return out[:, :_BANDS, :, :_HO]
